# Initial kernel scaffold; baseline (speedup 1.0000x reference)
import jax
import jax.numpy as jnp
from jax.experimental import pallas as pl


def kernel(ctx, table, tokens):
    raise NotImplementedError("write your pallas kernel here")


# speedup vs baseline: 1.2323x; 1.2323x over previous
import functools

import jax
import jax.numpy as jnp
from jax import lax
from jax.experimental import pallas as pl
from jax.experimental.pallas import tpu as pltpu
from jax.experimental.pallas import tpu_sc as plsc

N_CLS = 1000
N_CTX = 16
CTX_DIM = 512
CONTEXT_LEN = 77
N_SUFFIX = CONTEXT_LEN - 1 - N_CTX

NUM_WORKERS = 32
CPW = 32
IDXW = 80


def _sc_prompt_assemble(ctx2d, table, gidx):
    mesh = plsc.VectorSubcoreMesh(core_axis_name="c", subcore_axis_name="s")

    @functools.partial(
        pl.kernel,
        mesh=mesh,
        compiler_params=pltpu.CompilerParams(use_tc_tiling_on_sc=False),
        out_type=jax.ShapeDtypeStruct((N_CLS * CONTEXT_LEN, CTX_DIM), jnp.float32),
        scratch_types=[
            pltpu.VMEM((IDXW,), jnp.int32),
            pltpu.VMEM((8, CTX_DIM), jnp.float32),
            pltpu.VMEM((N_SUFFIX, CTX_DIM), jnp.float32),
            pltpu.VMEM((N_CTX, CTX_DIM), jnp.float32),
            pltpu.SemaphoreType.DMA,
        ],
    )
    def k(ctx_hbm, table_hbm, gidx_hbm, out_hbm, idx_v, pbuf, sbuf, cbuf, sem):
        wid = lax.axis_index("s") * 2 + lax.axis_index("c")

        def body(i, carry):
            c = wid * CPW + i

            @pl.when(c < N_CLS)
            def _():
                pltpu.sync_copy(gidx_hbm.at[c], idx_v)
                cp1 = pltpu.async_copy(
                    table_hbm.at[idx_v.at[pl.ds(8, N_SUFFIX)]],
                    sbuf,
                    sem,
                )
                cp2 = pltpu.async_copy(
                    table_hbm.at[idx_v.at[pl.ds(0, 1)]],
                    pbuf.at[pl.ds(0, 1)],
                    sem,
                )
                pltpu.sync_copy(ctx_hbm.at[pl.ds(c * N_CTX, N_CTX)], cbuf)
                cp1.wait()
                cp2.wait()
                base = c * CONTEXT_LEN
                pltpu.sync_copy(pbuf.at[pl.ds(0, 1)], out_hbm.at[pl.ds(base, 1)])
                pltpu.sync_copy(cbuf, out_hbm.at[pl.ds(base + 1, N_CTX)])
                pltpu.sync_copy(sbuf, out_hbm.at[pl.ds(base + 1 + N_CTX, N_SUFFIX)])

            return carry

        lax.fori_loop(0, CPW, body, 0)

    return k(ctx2d, table, gidx)


def kernel(ctx, table, tokens):
    gidx = jnp.zeros((N_CLS, IDXW), jnp.int32)
    gidx = gidx.at[:, 0].set(tokens[:, 0])
    gidx = gidx.at[:, 8 : 8 + N_SUFFIX].set(tokens[:, 1 + N_CTX :])
    ctx2d = ctx.reshape(N_CLS * N_CTX, CTX_DIM)
    out = _sc_prompt_assemble(ctx2d, table, gidx)
    return out.reshape(N_CLS, CONTEXT_LEN, CTX_DIM)


# speedup vs baseline: 4.0723x; 3.3047x over previous
import functools

import jax
import jax.numpy as jnp
from jax import lax
from jax.experimental import pallas as pl
from jax.experimental.pallas import tpu as pltpu
from jax.experimental.pallas import tpu_sc as plsc

N_CLS = 1000
N_CTX = 16
CTX_DIM = 512
CONTEXT_LEN = 77
N_SUFFIX = CONTEXT_LEN - 1 - N_CTX

NUM_WORKERS = 32
CPW = 32
GIDX_W = 64


def _sc_prompt_assemble(ctx, table, gidx):
    mesh = plsc.VectorSubcoreMesh(core_axis_name="c", subcore_axis_name="s")

    @functools.partial(
        pl.kernel,
        mesh=mesh,
        out_type=jax.ShapeDtypeStruct((N_CLS, CONTEXT_LEN, CTX_DIM), jnp.float32),
        scratch_types=[
            pltpu.VMEM((GIDX_W,), jnp.int32),
            pltpu.VMEM((GIDX_W, CTX_DIM), jnp.float32),
            pltpu.VMEM((N_CTX, CTX_DIM), jnp.float32),
            pltpu.SemaphoreType.DMA,
        ],
    )
    def k(ctx_hbm, table_hbm, gidx_hbm, out_hbm, idx_v, sbuf, cbuf, sem):
        wid = lax.axis_index("s") * 2 + lax.axis_index("c")
        lane = lax.iota(jnp.int32, 16)

        def body(i, carry):
            c = wid * CPW + i

            @pl.when(c < N_CLS)
            def _():
                pltpu.sync_copy(gidx_hbm.at[pl.ds(c * GIDX_W, GIDX_W)], idx_v)
                cp1 = pltpu.async_copy(table_hbm.at[idx_v], sbuf, sem)
                cp2 = pltpu.async_copy(ctx_hbm.at[c], cbuf, sem)
                cp1.wait()
                cp2.wait()
                out_c = out_hbm.at[c]
                pltpu.sync_copy(cbuf, out_c.at[lane + 1])
                for kk in range(4):
                    dst = 17 + kk * 16 + lane
                    if kk == 3:
                        dst = jnp.where(dst < CONTEXT_LEN, dst, 0)
                    pltpu.sync_copy(sbuf.at[pl.ds(kk * 16, 16)], out_c.at[dst])

            return carry

        lax.fori_loop(0, CPW, body, 0)

    return k(ctx, table, gidx)


def kernel(ctx, table, tokens):
    gidx = jnp.concatenate(
        [tokens[:, 1 + N_CTX :], jnp.repeat(tokens[:, :1], 4, axis=1)], axis=1
    ).reshape(-1)
    return _sc_prompt_assemble(ctx, table, gidx)


# speedup vs baseline: 4.7927x; 1.1769x over previous
import functools

import jax
import jax.numpy as jnp
from jax import lax
from jax.experimental import pallas as pl
from jax.experimental.pallas import tpu as pltpu
from jax.experimental.pallas import tpu_sc as plsc

N_CLS = 1000
N_CTX = 16
CTX_DIM = 512
CONTEXT_LEN = 77
N_SUFFIX = CONTEXT_LEN - 1 - N_CTX

NUM_WORKERS = 32
CPW = 32
GIDX_W = 64
BUF_ROWS = GIDX_W + N_CTX
NBUF = 3
NSLOTS = CPW + 1
GIDX_CLS = NUM_WORKERS * CPW + NSLOTS - CPW


def _sc_prompt_assemble(ctx, table, gidx, oidx):
    mesh = plsc.VectorSubcoreMesh(core_axis_name="c", subcore_axis_name="s")

    @functools.partial(
        pl.kernel,
        mesh=mesh,
        out_type=jax.ShapeDtypeStruct((N_CLS, CONTEXT_LEN, CTX_DIM), jnp.float32),
        scratch_types=[
            pltpu.VMEM((NSLOTS * GIDX_W,), jnp.int32),
            pltpu.VMEM((BUF_ROWS,), jnp.int32),
            pltpu.VMEM((BUF_ROWS, CTX_DIM), jnp.float32),
            pltpu.VMEM((BUF_ROWS, CTX_DIM), jnp.float32),
            pltpu.VMEM((BUF_ROWS, CTX_DIM), jnp.float32),
            pltpu.SemaphoreType.DMA,
            pltpu.SemaphoreType.DMA,
            pltpu.SemaphoreType.DMA,
            pltpu.SemaphoreType.DMA,
            pltpu.SemaphoreType.DMA,
            pltpu.SemaphoreType.DMA,
        ],
    )
    def k(ctx_hbm, table_hbm, gidx_hbm, oidx_hbm, out_hbm,
          idxall_v, oidx_v, buf0, buf1, buf2,
          gsem0, gsem1, gsem2, ssem0, ssem1, ssem2):
        bufs = (buf0, buf1, buf2)
        gsems = (gsem0, gsem1, gsem2)
        ssems = (ssem0, ssem1, ssem2)
        wid = lax.axis_index("s") * 2 + lax.axis_index("c")
        cbase = wid * CPW

        pltpu.sync_copy(oidx_hbm, oidx_v)
        pltpu.sync_copy(
            gidx_hbm.at[pl.ds(cbase * GIDX_W, NSLOTS * GIDX_W)], idxall_v
        )

        def valid(i):
            return jnp.logical_and(
                jnp.logical_and(i >= 0, i < CPW), cbase + i < N_CLS
            )

        def start(i, b):
            @pl.when(valid(i))
            def _():
                pltpu.async_copy(
                    table_hbm.at[idxall_v.at[pl.ds(i * GIDX_W, GIDX_W)]],
                    bufs[b].at[pl.ds(0, GIDX_W)],
                    gsems[b],
                )
                pltpu.async_copy(
                    ctx_hbm.at[cbase + i],
                    bufs[b].at[pl.ds(GIDX_W, N_CTX)],
                    gsems[b],
                )

        def finish(i, b):
            @pl.when(valid(i))
            def _():
                pltpu.make_async_copy(
                    table_hbm.at[idxall_v.at[pl.ds(i * GIDX_W, GIDX_W)]],
                    bufs[b].at[pl.ds(0, GIDX_W)],
                    gsems[b],
                ).wait()
                pltpu.make_async_copy(
                    ctx_hbm.at[cbase + i],
                    bufs[b].at[pl.ds(GIDX_W, N_CTX)],
                    gsems[b],
                ).wait()
                pltpu.async_copy(
                    bufs[b], out_hbm.at[cbase + i].at[oidx_v], ssems[b]
                )

        def drain(i, b):
            @pl.when(valid(i))
            def _():
                pltpu.make_async_copy(
                    bufs[b], out_hbm.at[cbase + i].at[oidx_v], ssems[b]
                ).wait()

        start(0, 0)

        def body(j, carry):
            for b in range(NBUF):
                s = NBUF * j + b

                @pl.when(s < NSLOTS)
                def _(s=s, b=b):
                    drain(s - 2, (b + 1) % NBUF)
                    start(s + 1, (b + 1) % NBUF)
                    finish(s - 1, (b + NBUF - 1) % NBUF)

            return carry

        nsteps = -(-NSLOTS // NBUF)
        lax.fori_loop(0, nsteps, body, 0)
        drain(CPW - 1, (CPW - 1) % NBUF)

    return k(ctx, table, gidx, oidx)


def kernel(ctx, table, tokens):
    gidx2d = jnp.concatenate(
        [tokens[:, 1 + N_CTX :], jnp.repeat(tokens[:, :1], 4, axis=1)], axis=1
    )
    gidx2d = jnp.pad(gidx2d, ((0, GIDX_CLS - N_CLS), (0, 0)))
    gidx = gidx2d.reshape(-1)
    oidx = jnp.concatenate(
        [
            jnp.arange(1 + N_CTX, CONTEXT_LEN, dtype=jnp.int32),
            jnp.zeros((4,), jnp.int32),
            jnp.arange(1, 1 + N_CTX, dtype=jnp.int32),
        ]
    )
    return _sc_prompt_assemble(ctx, table, gidx, oidx)


# speedup vs baseline: 8.3085x; 1.7336x over previous
import functools

import jax
import jax.numpy as jnp
from jax import lax
from jax.experimental import pallas as pl
from jax.experimental.pallas import tpu as pltpu
from jax.experimental.pallas import tpu_sc as plsc

N_CLS = 1000
N_CTX = 16
CTX_DIM = 512
CONTEXT_LEN = 77
N_SUFFIX = CONTEXT_LEN - 1 - N_CTX

NUM_WORKERS = 32
CB = 32
LAST_CBASE = N_CLS - CB
NBUF = 3

_A_TOKENS = [0] + list(range(1 + N_CTX, CONTEXT_LEN))
_B_TOKENS = list(range(1, 1 + N_CTX))
_SLOT_TOKENS = _A_TOKENS + _B_TOKENS
_CHUNKS = []
for _s in range(0, 60, 2):
    _CHUNKS.append(("table", _s))
_CHUNKS.append(("table1", 60))
for _s in range(61, 77, 2):
    _CHUNKS.append(("ctx", _s))
NCHUNK = len(_CHUNKS)


def _sc_prompt_assemble(ctx2d, table, cidx):
    mesh = plsc.VectorSubcoreMesh(core_axis_name="c", subcore_axis_name="s")

    @functools.partial(
        pl.kernel,
        mesh=mesh,
        out_type=jax.ShapeDtypeStruct((CONTEXT_LEN * N_CLS, CTX_DIM), jnp.float32),
        scratch_types=[
            pltpu.VMEM((CONTEXT_LEN * CB,), jnp.int32),
            pltpu.VMEM((2 * CB, CTX_DIM), jnp.float32),
            pltpu.VMEM((2 * CB, CTX_DIM), jnp.float32),
            pltpu.VMEM((2 * CB, CTX_DIM), jnp.float32),
            pltpu.SemaphoreType.DMA,
            pltpu.SemaphoreType.DMA,
            pltpu.SemaphoreType.DMA,
            pltpu.SemaphoreType.DMA,
            pltpu.SemaphoreType.DMA,
            pltpu.SemaphoreType.DMA,
        ],
    )
    def k(ctx_hbm, table_hbm, cidx_hbm, out_hbm,
          idx_v, buf0, buf1, buf2,
          gsem0, gsem1, gsem2, ssem0, ssem1, ssem2):
        bufs = (buf0, buf1, buf2)
        gsems = (gsem0, gsem1, gsem2)
        ssems = (ssem0, ssem1, ssem2)
        wid = lax.axis_index("s") * 2 + lax.axis_index("c")
        cbase = jnp.minimum(wid * CB, LAST_CBASE)

        pltpu.sync_copy(
            cidx_hbm.at[pl.ds(wid * (CONTEXT_LEN * CB), CONTEXT_LEN * CB)],
            idx_v,
        )

        def chunk_rows(ci):
            kind, s0 = _CHUNKS[ci]
            return CB if kind == "table1" else 2 * CB

        def src_ref(ci):
            kind, _ = _CHUNKS[ci]
            return ctx_hbm if kind == "ctx" else table_hbm

        def start(ci, b):
            n = chunk_rows(ci)
            _, s0 = _CHUNKS[ci]
            pltpu.async_copy(
                src_ref(ci).at[idx_v.at[pl.ds(s0 * CB, n)]],
                bufs[b].at[pl.ds(0, n)],
                gsems[b],
            )

        def finish(ci, b):
            n = chunk_rows(ci)
            kind, s0 = _CHUNKS[ci]
            pltpu.make_async_copy(
                src_ref(ci).at[idx_v.at[pl.ds(s0 * CB, n)]],
                bufs[b].at[pl.ds(0, n)],
                gsems[b],
            ).wait()
            for kk in range(n // CB):
                t = _SLOT_TOKENS[s0 + kk]
                pltpu.async_copy(
                    bufs[b].at[pl.ds(kk * CB, CB)],
                    out_hbm.at[pl.ds(t * N_CLS + cbase, CB)],
                    ssems[b],
                )

        def drain(ci, b):
            n = chunk_rows(ci)
            _, s0 = _CHUNKS[ci]
            for kk in range(n // CB):
                t = _SLOT_TOKENS[s0 + kk]
                pltpu.make_async_copy(
                    bufs[b].at[pl.ds(kk * CB, CB)],
                    out_hbm.at[pl.ds(t * N_CLS + cbase, CB)],
                    ssems[b],
                ).wait()

        for s in range(NCHUNK + 2):
            if s - 2 >= 0:
                drain(s - 2, (s - 2) % NBUF)
            if s < NCHUNK:
                start(s, s % NBUF)
            if 0 <= s - 1 < NCHUNK:
                finish(s - 1, (s - 1) % NBUF)

    return k(ctx2d, table, cidx)


def kernel(ctx, table, tokens):
    w = jnp.arange(NUM_WORKERS, dtype=jnp.int32)
    cs = jnp.minimum(w * CB, LAST_CBASE)[:, None] + jnp.arange(CB, dtype=jnp.int32)
    a_tok = jnp.asarray(_A_TOKENS, dtype=jnp.int32)
    a_idx = tokens[cs][:, :, a_tok].transpose(0, 2, 1)
    b_idx = cs[:, None, :] * N_CTX + jnp.arange(N_CTX, dtype=jnp.int32)[None, :, None]
    cidx = jnp.concatenate([a_idx, b_idx], axis=1).reshape(-1)
    ctx2d = ctx.reshape(N_CLS * N_CTX, CTX_DIM)
    out2d = _sc_prompt_assemble(ctx2d, table, cidx)
    return out2d.reshape(CONTEXT_LEN, N_CLS, CTX_DIM).transpose(1, 0, 2)


# speedup vs baseline: 8.3614x; 1.0064x over previous
import functools

import jax
import jax.numpy as jnp
from jax import lax
from jax.experimental import pallas as pl
from jax.experimental.pallas import tpu as pltpu
from jax.experimental.pallas import tpu_sc as plsc

N_CLS = 1000
N_CTX = 16
CTX_DIM = 512
CONTEXT_LEN = 77
N_SUFFIX = CONTEXT_LEN - 1 - N_CTX

NUM_WORKERS = 32
CB = 32
LAST_CBASE = N_CLS - CB
NBUF = 3

_A_TOKENS = [0] + list(range(1 + N_CTX, CONTEXT_LEN))
_B_TOKENS = list(range(1, 1 + N_CTX))
_SLOT_TOKENS = _A_TOKENS + _B_TOKENS
_CHUNKS = []
for _s in range(0, 60, 2):
    _CHUNKS.append(("table", _s))
_CHUNKS.append(("table1", 60))
for _s in range(61, 77, 2):
    _CHUNKS.append(("ctx", _s))
NCHUNK = len(_CHUNKS)


def _sc_prompt_assemble(ctx2d, table, tokens):
    mesh = plsc.VectorSubcoreMesh(core_axis_name="c", subcore_axis_name="s")

    @functools.partial(
        pl.kernel,
        mesh=mesh,
        out_type=jax.ShapeDtypeStruct((CONTEXT_LEN * N_CLS, CTX_DIM), jnp.float32),
        scratch_types=[
            pltpu.VMEM((CONTEXT_LEN * CB,), jnp.int32),
            pltpu.VMEM((2 * CB, CTX_DIM), jnp.float32),
            pltpu.VMEM((2 * CB, CTX_DIM), jnp.float32),
            pltpu.VMEM((2 * CB, CTX_DIM), jnp.float32),
            pltpu.SemaphoreType.DMA,
            pltpu.SemaphoreType.DMA,
            pltpu.SemaphoreType.DMA,
            pltpu.SemaphoreType.DMA,
            pltpu.SemaphoreType.DMA,
            pltpu.SemaphoreType.DMA,
        ],
    )
    def k(ctx_hbm, table_hbm, tokt_hbm, out_hbm,
          idx_v, buf0, buf1, buf2,
          gsem0, gsem1, gsem2, ssem0, ssem1, ssem2):
        bufs = (buf0, buf1, buf2)
        gsems = (gsem0, gsem1, gsem2)
        ssems = (ssem0, ssem1, ssem2)
        wid = lax.axis_index("s") * 2 + lax.axis_index("c")
        cbase = jnp.minimum(wid * CB, LAST_CBASE)

        idx_cps = [
            pltpu.async_copy(
                tokt_hbm.at[pl.ds(t * N_CLS + cbase, CB)],
                idx_v.at[pl.ds(s * CB, CB)],
                gsem0,
            )
            for s, t in enumerate(_A_TOKENS)
        ]
        lane = lax.iota(jnp.int32, 16)
        for s, t in enumerate(_SLOT_TOKENS):
            if s < len(_A_TOKENS):
                continue
            for h in range(2):
                vals = (cbase + lane + 16 * h) * N_CTX + (t - 1)
                idx_v[pl.ds(s * CB + 16 * h, 16)] = vals
        for cp in idx_cps:
            cp.wait()

        def chunk_rows(ci):
            kind, s0 = _CHUNKS[ci]
            return CB if kind == "table1" else 2 * CB

        def src_ref(ci):
            kind, _ = _CHUNKS[ci]
            return ctx_hbm if kind == "ctx" else table_hbm

        def start(ci, b):
            n = chunk_rows(ci)
            _, s0 = _CHUNKS[ci]
            pltpu.async_copy(
                src_ref(ci).at[idx_v.at[pl.ds(s0 * CB, n)]],
                bufs[b].at[pl.ds(0, n)],
                gsems[b],
            )

        def finish(ci, b):
            n = chunk_rows(ci)
            kind, s0 = _CHUNKS[ci]
            pltpu.make_async_copy(
                src_ref(ci).at[idx_v.at[pl.ds(s0 * CB, n)]],
                bufs[b].at[pl.ds(0, n)],
                gsems[b],
            ).wait()
            for kk in range(n // CB):
                t = _SLOT_TOKENS[s0 + kk]
                pltpu.async_copy(
                    bufs[b].at[pl.ds(kk * CB, CB)],
                    out_hbm.at[pl.ds(t * N_CLS + cbase, CB)],
                    ssems[b],
                )

        def drain(ci, b):
            n = chunk_rows(ci)
            _, s0 = _CHUNKS[ci]
            for kk in range(n // CB):
                t = _SLOT_TOKENS[s0 + kk]
                pltpu.make_async_copy(
                    bufs[b].at[pl.ds(kk * CB, CB)],
                    out_hbm.at[pl.ds(t * N_CLS + cbase, CB)],
                    ssems[b],
                ).wait()

        for s in range(NCHUNK + 2):
            if s - 2 >= 0:
                drain(s - 2, (s - 2) % NBUF)
            if s < NCHUNK:
                start(s, s % NBUF)
            if 0 <= s - 1 < NCHUNK:
                finish(s - 1, (s - 1) % NBUF)

    return k(ctx2d, table, tokens)


def kernel(ctx, table, tokens):
    ctx2d = ctx.reshape(N_CLS * N_CTX, CTX_DIM)
    tokens_t = tokens.T.reshape(-1)
    out2d = _sc_prompt_assemble(ctx2d, table, tokens_t)
    return out2d.reshape(CONTEXT_LEN, N_CLS, CTX_DIM).transpose(1, 0, 2)
